# baseline (speedup 1.0000x reference)
import functools

import jax
import jax.numpy as jnp
from jax import lax
from jax.experimental import pallas as pl
from jax.experimental.pallas import tpu as pltpu
from jax.experimental.pallas import tpu_sc as plsc

NUM_E = 8
D_MODEL = 2048
H0 = 512
H1 = 256
Y_DIM = 1024
ALPHA = 0.2
N_TOK = 8192

GB = 64
GL = 128
TM = 1024
T_TILES = N_TOK // TM

NC, NS = 2, 16
NW = NC * NS
ROWS_PER_W = N_TOK // NW
SC_CH = 16
SC_CG = 32


def _routing_body(g_ref, dst_ref, seg_ref, work_ref):
    g = jnp.clip(g_ref[...], 0, NUM_E - 1)
    rl = lax.broadcasted_iota(jnp.int32, (GL, GL), 0)
    cl = lax.broadcasted_iota(jnp.int32, (GL, GL), 1)
    up_l = (rl < cl).astype(jnp.float32)
    rb = lax.broadcasted_iota(jnp.int32, (GB, GB), 0)
    cb = lax.broadcasted_iota(jnp.int32, (GB, GB), 1)
    lo_b = (rb > cb).astype(jnp.float32)

    off = jnp.int32(0)
    dst = jnp.zeros((GB, GL), jnp.int32)
    for e in range(NUM_E):
        m = (g == e).astype(jnp.float32)
        p = jnp.dot(m, up_l, preferred_element_type=jnp.float32)
        s = jnp.sum(m, axis=1, keepdims=True)
        c = jnp.dot(lo_b, s, preferred_element_type=jnp.float32)
        cnt = jnp.sum(s).astype(jnp.int32)
        d_e = off + (c + p).astype(jnp.int32)
        dst = jnp.where(g == e, d_e, dst)
        seg_ref[0, e] = off
        off = off + cnt
        seg_ref[1, e] = off
    dst_ref[...] = dst

    def _clear(i, carry):
        work_ref[0, i] = 0
        work_ref[1, i] = 0
        return carry

    lax.fori_loop(0, N_ITEMS, _clear, 0)

    def _compact(i, k):
        e = i // T_TILES
        t = i - e * T_TILES
        s = seg_ref[0, e]
        en = seg_ref[1, e]
        hit = (s < (t + 1) * TM) & (en > t * TM)

        @pl.when(hit)
        def _():
            work_ref[0, k] = e
            work_ref[1, k] = t

        return k + hit.astype(jnp.int32)

    lax.fori_loop(0, NUM_E * T_TILES, _compact, 0)


def _route(gate):
    dst2d, seg, work = pl.pallas_call(
        _routing_body,
        out_shape=(
            jax.ShapeDtypeStruct((GB, GL), jnp.int32),
            jax.ShapeDtypeStruct((2, NUM_E), jnp.int32),
            jax.ShapeDtypeStruct((2, N_ITEMS), jnp.int32),
        ),
        in_specs=[pl.BlockSpec(memory_space=pltpu.VMEM)],
        out_specs=(
            pl.BlockSpec(memory_space=pltpu.VMEM),
            pl.BlockSpec(memory_space=pltpu.SMEM),
            pl.BlockSpec(memory_space=pltpu.SMEM),
        ),
    )(gate.reshape(GB, GL))
    return dst2d.reshape(N_TOK), seg, work


def _sc_scatter_body(h_hbm, dst3_hbm, hs_hbm,
                     hbuf0, hbuf1, iall, sem0, sem1):
    wid = lax.axis_index("s") * NC + lax.axis_index("c")
    base = wid * ROWS_PER_W
    bufs = ((hbuf0, sem0), (hbuf1, sem1))

    pltpu.sync_copy(dst3_hbm.at[wid], iall)
    descs = [None, None]
    for j in range(ROWS_PER_W // SC_CH):
        buf, sem = bufs[j % 2]
        if descs[j % 2] is not None:
            descs[j % 2].wait()
        pltpu.sync_copy(h_hbm.at[pl.ds(base + j * SC_CH, SC_CH)], buf)
        descs[j % 2] = pltpu.async_copy(buf, hs_hbm.at[iall.at[j]], sem)
    descs[0].wait()
    descs[1].wait()


def _sc_gather_body(ys_hbm, dst3_hbm, out_hbm,
                    ybuf0, ybuf1, iall, sem0, sem1):
    wid = lax.axis_index("s") * NC + lax.axis_index("c")
    base = wid * ROWS_PER_W
    bufs = ((ybuf0, sem0), (ybuf1, sem1))
    nch = ROWS_PER_W // SC_CG

    pltpu.sync_copy(dst3_hbm.at[wid], iall)
    prev = None
    for j in range(nch + 1):
        if j < nch:
            buf, sem = bufs[j % 2]
            b = base + j * SC_CG
            g = pltpu.async_copy(ys_hbm.at[iall.at[j]], buf, sem)
        if prev is not None:
            pg, pbuf, pb = prev
            pg.wait()
            pltpu.sync_copy(pbuf, out_hbm.at[pl.ds(pb, SC_CG)])
        prev = (g, buf, b) if j < nch else None


def _sc_mesh():
    return plsc.VectorSubcoreMesh(
        core_axis_name="c", subcore_axis_name="s",
        num_cores=NC, num_subcores=NS)


def _sc_scatter(h, dst):
    return pl.kernel(
        _sc_scatter_body,
        out_type=jax.ShapeDtypeStruct((N_TOK, D_MODEL), jnp.float32),
        mesh=_sc_mesh(),
        scratch_types=[
            pltpu.VMEM((SC_CH, D_MODEL), jnp.float32),
            pltpu.VMEM((SC_CH, D_MODEL), jnp.float32),
            pltpu.VMEM((ROWS_PER_W // SC_CH, SC_CH), jnp.int32),
            pltpu.SemaphoreType.DMA,
            pltpu.SemaphoreType.DMA,
        ],
    )(h, dst.reshape(NW, ROWS_PER_W // SC_CH, SC_CH))


def _sc_gather(ys, dst):
    return pl.kernel(
        _sc_gather_body,
        out_type=jax.ShapeDtypeStruct((N_TOK, Y_DIM), jnp.float32),
        mesh=_sc_mesh(),
        scratch_types=[
            pltpu.VMEM((SC_CG, Y_DIM), jnp.float32),
            pltpu.VMEM((SC_CG, Y_DIM), jnp.float32),
            pltpu.VMEM((ROWS_PER_W // SC_CG, SC_CG), jnp.int32),
            pltpu.SemaphoreType.DMA,
            pltpu.SemaphoreType.DMA,
        ],
    )(ys, dst.reshape(NW, ROWS_PER_W // SC_CG, SC_CG))


N_ITEMS = T_TILES + NUM_E - 1


def _mlp_body(seg_ref, work_ref,
              hs_ref, W0_ref, b0_ref, W1_ref, b1_ref, W2_ref, b2_ref,
              out_ref):
    i = pl.program_id(0)
    e = work_ref[0, i]
    t = work_ref[1, i]
    start = seg_ref[0, e]
    end = seg_ref[1, e]
    row0 = t * TM

    x = hs_ref[...]
    z = jnp.dot(x, W0_ref[0], preferred_element_type=jnp.float32)
    z = z + b0_ref[e]
    z = jnp.where(z >= 0, z, ALPHA * z)
    z = jnp.dot(z, W1_ref[e], preferred_element_type=jnp.float32)
    z = z + b1_ref[e]
    z = jnp.where(z >= 0, z, ALPHA * z)
    y = jnp.dot(z, W2_ref[e], preferred_element_type=jnp.float32)
    y = y + b2_ref[e]
    gid = row0 + lax.broadcasted_iota(jnp.int32, (TM, 1), 0)
    mask = (gid >= start) & (gid < end)
    out_ref[...] = jnp.where(mask, y, out_ref[...])


def _grouped_mlp(seg, work, hs, W0, b0, W1, b1, W2, b2):
    whole = lambda i, seg, wk: (0, 0, 0)
    grid_spec = pltpu.PrefetchScalarGridSpec(
        num_scalar_prefetch=2,
        grid=(N_ITEMS,),
        in_specs=[
            pl.BlockSpec((TM, D_MODEL), lambda i, seg, wk: (wk[1, i], 0)),
            pl.BlockSpec((1, D_MODEL, H0),
                         lambda i, seg, wk: (wk[0, i], 0, 0)),
            pl.BlockSpec((NUM_E, 1, H0), whole),
            pl.BlockSpec((NUM_E, H0, H1), whole),
            pl.BlockSpec((NUM_E, 1, H1), whole),
            pl.BlockSpec((NUM_E, H1, Y_DIM), whole),
            pl.BlockSpec((NUM_E, 1, Y_DIM), whole),
        ],
        out_specs=pl.BlockSpec((TM, Y_DIM), lambda i, seg, wk: (wk[1, i], 0)),
    )
    return pl.pallas_call(
        _mlp_body,
        grid_spec=grid_spec,
        out_shape=jax.ShapeDtypeStruct((N_TOK, Y_DIM), jnp.float32),
        compiler_params=pltpu.CompilerParams(
            dimension_semantics=("arbitrary",)),
    )(seg, work, hs, W0, b0[:, None, :], W1, b1[:, None, :],
      W2, b2[:, None, :])


@jax.jit
def kernel(h, gate_id, W0, b0, W1, b1, W2, b2):
    gate = gate_id.astype(jnp.int32)
    dst, seg, work = _route(gate)
    hs = _sc_scatter(h, dst)
    ys = _grouped_mlp(seg, work, hs, W0, b0, W1, b1, W2, b2)
    return _sc_gather(ys, dst)


# speedup vs baseline: 1.0142x; 1.0142x over previous
import functools

import jax
import jax.numpy as jnp
from jax import lax
from jax.experimental import pallas as pl
from jax.experimental.pallas import tpu as pltpu
from jax.experimental.pallas import tpu_sc as plsc

NUM_E = 8
D_MODEL = 2048
H0 = 512
H1 = 256
Y_DIM = 1024
ALPHA = 0.2
N_TOK = 8192

GB = 64
GL = 128
TM = 512
T_TILES = N_TOK // TM

NC, NS = 2, 16
NW = NC * NS
ROWS_PER_W = N_TOK // NW
SC_CH = 16
SC_CG = 32


def _routing_body(g_ref, dst_ref, seg_ref, work_ref):
    g = jnp.clip(g_ref[...], 0, NUM_E - 1)
    rl = lax.broadcasted_iota(jnp.int32, (GL, GL), 0)
    cl = lax.broadcasted_iota(jnp.int32, (GL, GL), 1)
    up_l = (rl < cl).astype(jnp.float32)
    rb = lax.broadcasted_iota(jnp.int32, (GB, GB), 0)
    cb = lax.broadcasted_iota(jnp.int32, (GB, GB), 1)
    lo_b = (rb > cb).astype(jnp.float32)

    off = jnp.int32(0)
    dst = jnp.zeros((GB, GL), jnp.int32)
    for e in range(NUM_E):
        m = (g == e).astype(jnp.float32)
        p = jnp.dot(m, up_l, preferred_element_type=jnp.float32)
        s = jnp.sum(m, axis=1, keepdims=True)
        c = jnp.dot(lo_b, s, preferred_element_type=jnp.float32)
        cnt = jnp.sum(s).astype(jnp.int32)
        d_e = off + (c + p).astype(jnp.int32)
        dst = jnp.where(g == e, d_e, dst)
        seg_ref[0, e] = off
        off = off + cnt
        seg_ref[1, e] = off
    dst_ref[...] = dst

    def _clear(i, carry):
        work_ref[0, i] = 0
        work_ref[1, i] = 0
        return carry

    lax.fori_loop(0, N_ITEMS, _clear, 0)

    def _compact(i, k):
        e = i // T_TILES
        t = i - e * T_TILES
        s = seg_ref[0, e]
        en = seg_ref[1, e]
        hit = (s < (t + 1) * TM) & (en > t * TM)

        @pl.when(hit)
        def _():
            work_ref[0, k] = e
            work_ref[1, k] = t

        return k + hit.astype(jnp.int32)

    lax.fori_loop(0, NUM_E * T_TILES, _compact, 0)


def _route(gate):
    dst2d, seg, work = pl.pallas_call(
        _routing_body,
        out_shape=(
            jax.ShapeDtypeStruct((GB, GL), jnp.int32),
            jax.ShapeDtypeStruct((2, NUM_E), jnp.int32),
            jax.ShapeDtypeStruct((2, N_ITEMS), jnp.int32),
        ),
        in_specs=[pl.BlockSpec(memory_space=pltpu.VMEM)],
        out_specs=(
            pl.BlockSpec(memory_space=pltpu.VMEM),
            pl.BlockSpec(memory_space=pltpu.SMEM),
            pl.BlockSpec(memory_space=pltpu.SMEM),
        ),
    )(gate.reshape(GB, GL))
    return dst2d.reshape(N_TOK), seg, work


def _sc_scatter_body(h_hbm, dst3_hbm, hs_hbm,
                     hbuf0, hbuf1, iall, sem0, sem1):
    wid = lax.axis_index("s") * NC + lax.axis_index("c")
    base = wid * ROWS_PER_W
    bufs = ((hbuf0, sem0), (hbuf1, sem1))

    pltpu.sync_copy(dst3_hbm.at[wid], iall)
    descs = [None, None]
    for j in range(ROWS_PER_W // SC_CH):
        buf, sem = bufs[j % 2]
        if descs[j % 2] is not None:
            descs[j % 2].wait()
        pltpu.sync_copy(h_hbm.at[pl.ds(base + j * SC_CH, SC_CH)], buf)
        descs[j % 2] = pltpu.async_copy(buf, hs_hbm.at[iall.at[j]], sem)
    descs[0].wait()
    descs[1].wait()


def _sc_gather_body(ys_hbm, dst3_hbm, out_hbm,
                    ybuf0, ybuf1, iall, sem0, sem1):
    wid = lax.axis_index("s") * NC + lax.axis_index("c")
    base = wid * ROWS_PER_W
    bufs = ((ybuf0, sem0), (ybuf1, sem1))
    nch = ROWS_PER_W // SC_CG

    pltpu.sync_copy(dst3_hbm.at[wid], iall)
    prev = None
    for j in range(nch + 1):
        if j < nch:
            buf, sem = bufs[j % 2]
            b = base + j * SC_CG
            g = pltpu.async_copy(ys_hbm.at[iall.at[j]], buf, sem)
        if prev is not None:
            pg, pbuf, pb = prev
            pg.wait()
            pltpu.sync_copy(pbuf, out_hbm.at[pl.ds(pb, SC_CG)])
        prev = (g, buf, b) if j < nch else None


def _sc_mesh():
    return plsc.VectorSubcoreMesh(
        core_axis_name="c", subcore_axis_name="s",
        num_cores=NC, num_subcores=NS)


def _sc_scatter(h, dst):
    return pl.kernel(
        _sc_scatter_body,
        out_type=jax.ShapeDtypeStruct((N_TOK, D_MODEL), jnp.float32),
        mesh=_sc_mesh(),
        scratch_types=[
            pltpu.VMEM((SC_CH, D_MODEL), jnp.float32),
            pltpu.VMEM((SC_CH, D_MODEL), jnp.float32),
            pltpu.VMEM((ROWS_PER_W // SC_CH, SC_CH), jnp.int32),
            pltpu.SemaphoreType.DMA,
            pltpu.SemaphoreType.DMA,
        ],
    )(h, dst.reshape(NW, ROWS_PER_W // SC_CH, SC_CH))


def _sc_gather(ys, dst):
    return pl.kernel(
        _sc_gather_body,
        out_type=jax.ShapeDtypeStruct((N_TOK, Y_DIM), jnp.float32),
        mesh=_sc_mesh(),
        scratch_types=[
            pltpu.VMEM((SC_CG, Y_DIM), jnp.float32),
            pltpu.VMEM((SC_CG, Y_DIM), jnp.float32),
            pltpu.VMEM((ROWS_PER_W // SC_CG, SC_CG), jnp.int32),
            pltpu.SemaphoreType.DMA,
            pltpu.SemaphoreType.DMA,
        ],
    )(ys, dst.reshape(NW, ROWS_PER_W // SC_CG, SC_CG))


N_ITEMS = T_TILES + NUM_E - 1


def _mlp_body(seg_ref, work_ref,
              hs_ref, W0_ref, b0_ref, W1_ref, b1_ref, W2_ref, b2_ref,
              out_ref):
    i = pl.program_id(0)
    e = work_ref[0, i]
    t = work_ref[1, i]
    start = seg_ref[0, e]
    end = seg_ref[1, e]
    row0 = t * TM

    x = hs_ref[...]
    z = jnp.dot(x, W0_ref[0], preferred_element_type=jnp.float32)
    z = z + b0_ref[e]
    z = jnp.where(z >= 0, z, ALPHA * z)
    z = jnp.dot(z, W1_ref[e], preferred_element_type=jnp.float32)
    z = z + b1_ref[e]
    z = jnp.where(z >= 0, z, ALPHA * z)
    y = jnp.dot(z, W2_ref[e], preferred_element_type=jnp.float32)
    y = y + b2_ref[e]
    gid = row0 + lax.broadcasted_iota(jnp.int32, (TM, 1), 0)
    mask = (gid >= start) & (gid < end)
    out_ref[...] = jnp.where(mask, y, out_ref[...])


def _grouped_mlp(seg, work, hs, W0, b0, W1, b1, W2, b2):
    whole = lambda i, seg, wk: (0, 0, 0)
    grid_spec = pltpu.PrefetchScalarGridSpec(
        num_scalar_prefetch=2,
        grid=(N_ITEMS,),
        in_specs=[
            pl.BlockSpec((TM, D_MODEL), lambda i, seg, wk: (wk[1, i], 0)),
            pl.BlockSpec((1, D_MODEL, H0),
                         lambda i, seg, wk: (wk[0, i], 0, 0)),
            pl.BlockSpec((NUM_E, 1, H0), whole),
            pl.BlockSpec((NUM_E, H0, H1), whole),
            pl.BlockSpec((NUM_E, 1, H1), whole),
            pl.BlockSpec((NUM_E, H1, Y_DIM), whole),
            pl.BlockSpec((NUM_E, 1, Y_DIM), whole),
        ],
        out_specs=pl.BlockSpec((TM, Y_DIM), lambda i, seg, wk: (wk[1, i], 0)),
    )
    return pl.pallas_call(
        _mlp_body,
        grid_spec=grid_spec,
        out_shape=jax.ShapeDtypeStruct((N_TOK, Y_DIM), jnp.float32),
        compiler_params=pltpu.CompilerParams(
            dimension_semantics=("arbitrary",)),
    )(seg, work, hs, W0, b0[:, None, :], W1, b1[:, None, :],
      W2, b2[:, None, :])


@jax.jit
def kernel(h, gate_id, W0, b0, W1, b1, W2, b2):
    gate = gate_id.astype(jnp.int32)
    dst, seg, work = _route(gate)
    hs = _sc_scatter(h, dst)
    ys = _grouped_mlp(seg, work, hs, W0, b0, W1, b1, W2, b2)
    return _sc_gather(ys, dst)
